# baseline (speedup 1.0000x reference)
import jax
import jax.numpy as jnp
from jax import lax
from jax.experimental import pallas as pl
from jax.experimental.pallas import tpu as pltpu
from jax.experimental.pallas import tpu_sc as plsc


def _crelu(x, leak=0.1):
    c = jnp.clip(x, 0.0, 127.0 / 128.0)
    return c + leak * (x - c)


def _sc_gather(table, indices):
    n = indices.shape[0]
    d = table.shape[1]
    w = 128
    assert n % w == 0
    mesh = plsc.VectorSubcoreMesh(core_axis_name="core", subcore_axis_name="subcore")
    idx2 = indices.reshape(1, n)

    @pl.kernel(
        out_type=jax.ShapeDtypeStruct((n, d), table.dtype),
        mesh=mesh,
    )
    def gather_kernel(table_hbm, idx_hbm, out_hbm):
        def body(idx_vmem, out_vmem):
            pltpu.sync_copy(table_hbm.at[idx_vmem.at[0]], out_vmem)

        pltpu.emit_pipeline(
            body,
            grid=(n // w,),
            in_specs=[pl.BlockSpec((1, w), index_map=lambda i: (0, i))],
            out_specs=[pl.BlockSpec((w, d), index_map=lambda i: (i, 0))],
            core_axis_name=("core", "subcore"),
            dimension_semantics=(pltpu.PARALLEL,),
        )(idx_hbm, out_hbm)

    return gather_kernel(table, idx2)


def _heads_tc(accum, whichrow, main_bias2d, w1r, selcat, w2flat, g1, g1t):
    b, d = accum.shape
    n_nets = selcat.shape[0]
    h1w = g1.shape[1]
    bs = 2048
    assert b % bs == 0

    def body(acc_ref, wm_ref, mb_ref, w1_ref, sc_ref, w2_ref, g1_ref, g1t_ref,
             out_ref):
        a = acc_ref[...] + mb_ref[...]
        psqt = a[:, :1]
        e = _crelu(a)
        wm = jnp.transpose(wm_ref[...], (1, 0)).astype(jnp.int32)
        onehot = (wm == lax.broadcasted_iota(jnp.int32, (bs, n_nets), 1)
                  ).astype(jnp.float32)
        selall = jnp.dot(onehot, sc_ref[...],
                         preferred_element_type=jnp.float32)
        b1sel = selall[:, 0:16]
        b2sel = selall[:, 32:64]
        w3sel = selall[:, 64:96]
        b3sel = selall[:, 96:97]
        sel = selall[:, 128:128 + n_nets * h1w]

        p1 = lax.dot_general(e.astype(jnp.bfloat16), w1_ref[...],
                             (((1,), (1,)), ((), ())),
                             preferred_element_type=jnp.float32)
        h1 = _crelu(jnp.dot(p1 * sel, g1_ref[...],
                            preferred_element_type=jnp.float32) + b1sel)
        q = jnp.dot(h1, g1t_ref[...], preferred_element_type=jnp.float32) * sel
        h2 = _crelu(jnp.dot(q, w2_ref[...],
                            preferred_element_type=jnp.float32) + b2sel)
        value = jnp.sum(h2 * w3sel, axis=1, keepdims=True) + b3sel
        out_ref[...] = jnp.tanh(jnp.reshape(value + psqt, (1, bs)))

    full = lambda shape: pl.BlockSpec(shape, lambda i: (0, 0))
    return pl.pallas_call(
        body,
        grid=(b // bs,),
        in_specs=[
            pl.BlockSpec((bs, d), lambda i: (i, 0)),
            pl.BlockSpec((1, bs), lambda i: (0, i)),
            full((1, d)),
            full(w1r.shape),
            full(selcat.shape),
            full(w2flat.shape),
            full(g1.shape),
            full(g1t.shape),
        ],
        out_specs=pl.BlockSpec((1, bs), lambda i: (0, i)),
        out_shape=jax.ShapeDtypeStruct((1, b), jnp.float32),
    )(accum, whichrow, main_bias2d, w1r, selcat, w2flat, g1, g1t)


def kernel(indices, offsets, which_model, lengths, table, main_bias, W1s, b1s,
           W2s, b2s, W3s, b3s):
    del offsets, lengths
    b = indices.shape[0]
    n_nets, h1w, d = W1s.shape
    h2w = W2s.shape[1]

    w1r = W1s.reshape(n_nets * h1w, d).astype(jnp.bfloat16)
    w2flat = jnp.transpose(W2s, (0, 2, 1)).reshape(n_nets * h1w, h2w)
    expand = (jnp.arange(n_nets)[:, None] ==
              jnp.arange(n_nets * h1w)[None, :] // h1w).astype(jnp.float32)
    selcat = jnp.concatenate([
        b1s, jnp.zeros((n_nets, 32 - h1w), jnp.float32),
        b2s.reshape(n_nets, h2w), W3s.reshape(n_nets, h2w),
        b3s.reshape(n_nets, 1), jnp.zeros((n_nets, 31), jnp.float32),
        expand,
    ], axis=1)
    g1 = (jnp.arange(n_nets * h1w)[:, None] % h1w ==
          jnp.arange(h1w)[None, :]).astype(jnp.float32)
    g1t = g1.T

    idx = indices.astype(jnp.int32)
    whichrow = which_model.astype(jnp.float32).reshape(1, b)
    mb2d = main_bias.reshape(1, d)

    accum = _sc_gather(table, idx)
    out = _heads_tc(accum, whichrow, mb2d, w1r, selcat, w2flat, g1, g1t)
    return out.reshape(b, 1)


# speedup vs baseline: 1.3479x; 1.3479x over previous
import jax
import jax.numpy as jnp
from jax import lax
from jax.experimental import pallas as pl
from jax.experimental.pallas import tpu as pltpu
from jax.experimental.pallas import tpu_sc as plsc


def _crelu(x, leak=0.1):
    c = jnp.clip(x, 0.0, 127.0 / 128.0)
    return c + leak * (x - c)


def _sc_gather(table, indices):
    n = indices.shape[0]
    d = table.shape[1]
    w = 128
    assert n % w == 0
    mesh = plsc.VectorSubcoreMesh(core_axis_name="core", subcore_axis_name="subcore")
    idx2 = indices.reshape(1, n)

    @pl.kernel(
        out_type=jax.ShapeDtypeStruct((n, d), table.dtype),
        mesh=mesh,
    )
    def gather_kernel(table_hbm, idx_hbm, out_hbm):
        def body(idx_vmem, out_vmem):
            pltpu.sync_copy(table_hbm.at[idx_vmem.at[0]], out_vmem)

        pltpu.emit_pipeline(
            body,
            grid=(n // w,),
            in_specs=[pl.BlockSpec((1, w), index_map=lambda i: (0, i))],
            out_specs=[pl.BlockSpec((w, d), index_map=lambda i: (i, 0))],
            core_axis_name=("core", "subcore"),
            dimension_semantics=(pltpu.PARALLEL,),
        )(idx_hbm, out_hbm)

    return gather_kernel(table, idx2)


def _heads_tc(accum, whichrow, main_bias2d, w1r, selcat, expandb, w2flat,
              g1, g1t):
    b, d = accum.shape
    n_nets = selcat.shape[0]
    h1w = g1.shape[1]
    bs = 4096
    ns = 512
    assert b % bs == 0 and bs % ns == 0

    def chain(a, wmrow, sc_ref, w1_ref, w2_ref, g1_ref, g1t_ref):
        psqt = a[:, :1]
        e = _crelu(a)
        wm = jnp.transpose(wmrow, (1, 0)).astype(jnp.int32)
        onehot = (wm == lax.broadcasted_iota(jnp.int32, (ns, n_nets), 1)
                  ).astype(jnp.float32)
        colhead = lax.broadcasted_iota(jnp.int32, (ns, n_nets * h1w), 1) // h1w
        sel = (colhead == wm).astype(jnp.float32)

        p1 = lax.dot_general(e.astype(jnp.bfloat16), w1_ref[...],
                             (((1,), (1,)), ((), ())),
                             preferred_element_type=jnp.float32)
        b1sel = jnp.dot(onehot, sc_ref[:, 0:16],
                        preferred_element_type=jnp.float32)
        b2sel = jnp.dot(onehot, sc_ref[:, 32:64],
                        preferred_element_type=jnp.float32)
        w3sel = jnp.dot(onehot, sc_ref[:, 64:96],
                        preferred_element_type=jnp.float32)
        b3sel = jnp.dot(onehot, sc_ref[:, 96:97],
                        preferred_element_type=jnp.float32)
        h1 = _crelu(jnp.dot(p1 * sel, g1_ref[...],
                            preferred_element_type=jnp.float32) + b1sel)
        q = jnp.dot(h1, g1t_ref[...], preferred_element_type=jnp.float32) * sel
        h2 = _crelu(jnp.dot(q, w2_ref[...],
                            preferred_element_type=jnp.float32) + b2sel)
        value = jnp.sum(h2 * w3sel, axis=1, keepdims=True) + b3sel
        return jnp.tanh(jnp.reshape(value + psqt, (1, ns)))

    def body(acc_ref, wm_ref, mb_ref, w1_ref, sc_ref, ex_ref, w2_ref, g1_ref,
             g1t_ref, out_ref):
        del ex_ref
        mb = mb_ref[...]
        outs = [
            chain(acc_ref[c * ns:(c + 1) * ns, :] + mb,
                  wm_ref[:, c * ns:(c + 1) * ns],
                  sc_ref, w1_ref, w2_ref, g1_ref, g1t_ref)
            for c in range(bs // ns)
        ]
        out_ref[...] = jnp.concatenate(outs, axis=1)

    full = lambda shape: pl.BlockSpec(shape, lambda i: (0, 0))
    return pl.pallas_call(
        body,
        grid=(b // bs,),
        in_specs=[
            pl.BlockSpec((bs, d), lambda i: (i, 0)),
            pl.BlockSpec((1, bs), lambda i: (0, i)),
            full((1, d)),
            full(w1r.shape),
            full(selcat.shape),
            full(expandb.shape),
            full(w2flat.shape),
            full(g1.shape),
            full(g1t.shape),
        ],
        out_specs=pl.BlockSpec((1, bs), lambda i: (0, i)),
        out_shape=jax.ShapeDtypeStruct((1, b), jnp.float32),
    )(accum, whichrow, main_bias2d, w1r, selcat, expandb, w2flat, g1, g1t)


def kernel(indices, offsets, which_model, lengths, table, main_bias, W1s, b1s,
           W2s, b2s, W3s, b3s):
    del offsets, lengths
    b = indices.shape[0]
    n_nets, h1w, d = W1s.shape
    h2w = W2s.shape[1]

    w1r = W1s.reshape(n_nets * h1w, d).astype(jnp.bfloat16)
    w2flat = jnp.transpose(W2s, (0, 2, 1)).reshape(n_nets * h1w, h2w)
    selcat = jnp.concatenate([
        b1s, jnp.zeros((n_nets, 32 - h1w), jnp.float32),
        b2s.reshape(n_nets, h2w), W3s.reshape(n_nets, h2w),
        b3s.reshape(n_nets, 1), jnp.zeros((n_nets, 31), jnp.float32),
    ], axis=1)
    expandb = (jnp.arange(n_nets)[:, None] ==
               jnp.arange(n_nets * h1w)[None, :] // h1w).astype(jnp.bfloat16)
    g1 = (jnp.arange(n_nets * h1w)[:, None] % h1w ==
          jnp.arange(h1w)[None, :]).astype(jnp.float32)
    g1t = g1.T

    idx = indices.astype(jnp.int32)
    whichrow = which_model.astype(jnp.float32).reshape(1, b)
    mb2d = main_bias.reshape(1, d)

    accum = _sc_gather(table, idx)
    out = _heads_tc(accum, whichrow, mb2d, w1r, selcat, expandb, w2flat,
                    g1, g1t)
    return out.reshape(b, 1)
